# baseline (speedup 1.0000x reference)
import functools

import jax
import jax.numpy as jnp
from jax import lax
from jax.experimental import pallas as pl
from jax.experimental.pallas import tpu as pltpu
from jax.experimental.pallas import tpu_sc as plsc

DIM = 64
BATCH = 4096
SEQ = 200
VOCAB = 1000000

NC = 2
NS = 16
NW = NC * NS
BTILE = BATCH // NW
NBUF = 4

_mesh = plsc.VectorSubcoreMesh(core_axis_name="c", subcore_axis_name="s")

NBLK = VOCAB // 128
BLK_PER_W = (NBLK + NW - 1) // NW


@functools.partial(
    pl.kernel,
    mesh=_mesh,
    out_type=jax.ShapeDtypeStruct((VOCAB // 2, 2 * DIM), jnp.float32),
    scratch_types=[
        *[pltpu.VMEM((DIM, 128), jnp.float32) for _ in range(2)],
        *[pltpu.VMEM((DIM, 128), jnp.float32) for _ in range(2)],
        pltpu.VMEM((32, 128), jnp.float32),
        *[pltpu.SemaphoreType.DMA for _ in range(2)],
        *[pltpu.SemaphoreType.DMA for _ in range(2)],
    ],
    compiler_params=pltpu.CompilerParams(
        use_tc_tiling_on_sc=True, needs_layout_passes=False),
)
def _relayout(tT_hbm, tail_hbm, tpair_hbm, a0, a1, b0, b1, tailbuf,
              gi0, gi1, go0, go1):
    ains = [a0, a1]
    bouts = [b0, b1]
    gis = [gi0, gi1]
    gos = [go0, go1]

    wid = lax.axis_index("s") * NC + lax.axis_index("c")

    @pl.when(wid == 0)
    def _():
        pltpu.sync_copy(tail_hbm, tailbuf)
        pltpu.sync_copy(tailbuf, tpair_hbm.at[pl.ds(NBLK * 64, 32), :])

    def in_slice(c):
        return tT_hbm.at[:, pl.ds(c * 128, 128)]

    def out_slice(c):
        return tpair_hbm.at[pl.ds(c * 64, 64), :]

    def start_in(c, buf, sem):
        pltpu.async_copy(in_slice(c), buf, sem)

    def wait_in(c, buf, sem):
        pltpu.make_async_copy(in_slice(c), buf, sem).wait()

    def start_out(c, bufp, sem):
        pltpu.async_copy(bufp, out_slice(c), sem)

    def wait_out(c, bufp, sem):
        pltpu.make_async_copy(bufp, out_slice(c), sem).wait()

    def transpose_blk(buf, bufp):
        i16 = lax.iota(jnp.int32, 16)

        @pl.loop(0, 8)
        def _(j0):
            jv = i16 + j0 * 16
            qv = lax.shift_right_logical(jv, 1)
            hv = lax.shift_left(lax.bitwise_and(jv, 1), 6)
            for k in range(16):
                rot = lax.bitwise_and(i16 + k, 15)
                for d0 in range(DIM // 16):
                    dv = rot + d0 * 16
                    val = plsc.load_gather(buf, [dv, jv])
                    plsc.store_scatter(bufp, [qv, hv + dv], val)

    def blk(i):
        return wid + NW * i

    for j in range(2):
        @pl.when(blk(j) < NBLK)
        def _():
            start_in(blk(j), ains[j], gis[j])

    @pl.loop(0, BLK_PER_W)
    def _loop(i):
        j = lax.bitwise_and(i, 1)
        for par in range(2):
            @pl.when(j == par)
            def _():
                c = blk(i)

                @pl.when(c < NBLK)
                def _():
                    wait_in(c, ains[par], gis[par])

                    @pl.when(i >= 2)
                    def _():
                        wait_out(blk(i - 2), bouts[par], gos[par])

                    transpose_blk(ains[par], bouts[par])
                    start_out(c, bouts[par], gos[par])

                    @pl.when(blk(i + 2) < NBLK)
                    def _():
                        start_in(blk(i + 2), ains[par], gis[par])

    for i_last in (BLK_PER_W - 2, BLK_PER_W - 1):
        par = i_last % 2

        @pl.when(blk(i_last) < NBLK)
        def _():
            wait_out(blk(i_last), bouts[par], gos[par])


@functools.partial(
    pl.kernel,
    mesh=_mesh,
    out_type=jax.ShapeDtypeStruct((SEQ, DIM, BATCH), jnp.float32),
    scratch_types=[
        pltpu.VMEM((SEQ, BTILE), jnp.int32),
        *[pltpu.VMEM((BTILE, 2 * DIM), jnp.float32) for _ in range(NBUF)],
        *[pltpu.VMEM((BTILE,), jnp.int32) for _ in range(NBUF)],
        *[pltpu.VMEM((BTILE,), jnp.int32) for _ in range(NBUF)],
        *[pltpu.VMEM((DIM, BTILE), jnp.float32) for _ in range(2)],
        *[pltpu.SemaphoreType.DMA for _ in range(NBUF)],
        *[pltpu.SemaphoreType.DMA for _ in range(2)],
    ],
    compiler_params=pltpu.CompilerParams(
        use_tc_tiling_on_sc=True, needs_layout_passes=False),
)
def _emb(idxT_hbm, tpair_hbm, out_hbm, idxblk,
         b0, b1, b2, b3, p0, p1, p2, p3, h0, h1, h2, h3, t0, t1,
         g0, g1, g2, g3, o0, o1):
    bufs = [b0, b1, b2, b3]
    pids = [p0, p1, p2, p3]
    hofs = [h0, h1, h2, h3]
    touts = [t0, t1]
    gsems = [g0, g1, g2, g3]
    osems = [o0, o1]

    wid = lax.axis_index("s") * NC + lax.axis_index("c")
    base_b = wid * BTILE

    pltpu.sync_copy(idxT_hbm.at[:, pl.ds(base_b, BTILE)], idxblk)

    def prep(u, pid, hof):
        @pl.loop(0, BTILE // 16)
        def _(g):
            v = idxblk[u, pl.ds(g * 16, 16)]
            pid[pl.ds(g * 16, 16)] = lax.shift_right_logical(v, 1)
            hof[pl.ds(g * 16, 16)] = lax.shift_left(lax.bitwise_and(v, 1), 6)

    def start_gather(buf, pid, sem):
        pltpu.async_copy(tpair_hbm.at[pid], buf, sem)

    def wait_gather(buf, pid, sem):
        pltpu.make_async_copy(tpair_hbm.at[pid], buf, sem).wait()

    def extract(buf, hof, tout):
        @pl.loop(0, BTILE // 16)
        def _(g):
            i16 = lax.iota(jnp.int32, 16)
            t16 = i16 + g * 16
            h16 = hof[pl.ds(g * 16, 16)]
            for k in range(16):
                rot = lax.bitwise_and(i16 + k, 15)
                hrot = h16 + rot
                for dblk in range(DIM // 16):
                    val = plsc.load_gather(buf, [t16, hrot + dblk * 16])
                    plsc.store_scatter(tout, [rot + dblk * 16, t16], val)

    def out_slab(u):
        return out_hbm.at[u, :, pl.ds(base_b, BTILE)]

    def start_scatter(u, tout, sem):
        pltpu.async_copy(tout, out_slab(u), sem)

    def wait_scatter(u, tout, sem):
        pltpu.make_async_copy(tout, out_slab(u), sem).wait()

    for j in range(NBUF):
        prep(j, pids[j], hofs[j])
        start_gather(bufs[j], pids[j], gsems[j])

    @pl.loop(0, SEQ, step=NBUF)
    def _ring(s):
        for j in range(NBUF):
            u = s + j
            wait_gather(bufs[j], pids[j], gsems[j])

            @pl.when(u >= 2)
            def _():
                wait_scatter(u - 2, touts[j % 2], osems[j % 2])

            extract(bufs[j], hofs[j], touts[j % 2])
            start_scatter(u, touts[j % 2], osems[j % 2])

            @pl.when(u + NBUF < SEQ)
            def _():
                prep(u + NBUF, pids[j], hofs[j])
                start_gather(bufs[j], pids[j], gsems[j])

    wait_scatter(SEQ - 2, touts[0], osems[0])
    wait_scatter(SEQ - 1, touts[1], osems[1])


def kernel(idx, table):
    idxT = jnp.transpose(idx)
    tail = jax.lax.slice(table, (VOCAB - 64, 0), (VOCAB, DIM))
    tpair = _relayout(jnp.transpose(table),
                      jnp.reshape(tail, (32, 2 * DIM)))
    kout = _emb(idxT, tpair)
    return jnp.transpose(kout, (2, 0, 1))


# speedup vs baseline: 1.1541x; 1.1541x over previous
import functools

import jax
import jax.numpy as jnp
from jax import lax
from jax.experimental import pallas as pl
from jax.experimental.pallas import tpu as pltpu
from jax.experimental.pallas import tpu_sc as plsc

DIM = 64
BATCH = 4096
SEQ = 200
VOCAB = 1000000

NC = 2
NS = 16
NW = NC * NS
BTILE = BATCH // NW
NBUF = 4
NTOK = BATCH * SEQ

_mesh = plsc.VectorSubcoreMesh(core_axis_name="c", subcore_axis_name="s")


@functools.partial(
    pl.kernel,
    mesh=_mesh,
    out_type=jax.ShapeDtypeStruct((NTOK, DIM), jnp.float32),
    scratch_types=[
        pltpu.VMEM((SEQ, BTILE), jnp.int32),
        *[pltpu.VMEM((BTILE, DIM), jnp.float32) for _ in range(NBUF)],
        *[pltpu.VMEM((BTILE,), jnp.int32) for _ in range(NBUF)],
        *[pltpu.SemaphoreType.DMA for _ in range(NBUF)],
        *[pltpu.SemaphoreType.DMA for _ in range(NBUF)],
    ],
    compiler_params=pltpu.CompilerParams(
        use_tc_tiling_on_sc=False, needs_layout_passes=False),
)
def _gather(idxT_hbm, table_hbm, rows_hbm, idxblk,
            b0, b1, b2, b3, p0, p1, p2, p3,
            g0, g1, g2, g3, o0, o1, o2, o3):
    bufs = [b0, b1, b2, b3]
    pids = [p0, p1, p2, p3]
    gsems = [g0, g1, g2, g3]
    osems = [o0, o1, o2, o3]

    wid = lax.axis_index("s") * NC + lax.axis_index("c")
    base_b = wid * BTILE

    pltpu.sync_copy(idxT_hbm.at[:, pl.ds(base_b, BTILE)], idxblk)

    def prep(u, pid):
        @pl.loop(0, BTILE // 16)
        def _(g):
            pid[pl.ds(g * 16, 16)] = idxblk[u, pl.ds(g * 16, 16)]

    def start_gather(buf, pid, sem):
        pltpu.async_copy(table_hbm.at[pid], buf, sem)

    def wait_gather(buf, pid, sem):
        pltpu.make_async_copy(table_hbm.at[pid], buf, sem).wait()

    def out_slab(u):
        return rows_hbm.at[pl.ds(u * BATCH + base_b, BTILE), :]

    def start_out(u, buf, sem):
        pltpu.async_copy(buf, out_slab(u), sem)

    def wait_out(u, buf, sem):
        pltpu.make_async_copy(buf, out_slab(u), sem).wait()

    for j in range(NBUF):
        prep(j, pids[j])
        start_gather(bufs[j], pids[j], gsems[j])

    @pl.loop(0, SEQ, step=NBUF)
    def _ring(s):
        for j in range(NBUF):
            u = s + j
            wait_gather(bufs[j], pids[j], gsems[j])
            start_out(u, bufs[j], osems[j])

            @pl.when(u + NBUF < SEQ)
            def _():
                prep(u + NBUF, pids[j])
                wait_out(u, bufs[j], osems[j])
                start_gather(bufs[j], pids[j], gsems[j])

    for j in range(NBUF):
        wait_out(SEQ - NBUF + j, bufs[j], osems[j])


@functools.partial(
    pl.kernel,
    mesh=_mesh,
    out_type=jax.ShapeDtypeStruct((SEQ, DIM, BATCH), jnp.float32),
    scratch_types=[
        *[pltpu.VMEM((BTILE // 2, 2 * DIM), jnp.float32) for _ in range(2)],
        *[pltpu.VMEM((DIM, BTILE), jnp.float32) for _ in range(2)],
        *[pltpu.SemaphoreType.DMA for _ in range(2)],
        *[pltpu.SemaphoreType.DMA for _ in range(2)],
    ],
    compiler_params=pltpu.CompilerParams(
        use_tc_tiling_on_sc=True, needs_layout_passes=False),
)
def _transpose(rows_hbm, out_hbm, a0, a1, t0, t1, gi0, gi1, go0, go1):
    ains = [a0, a1]
    touts = [t0, t1]
    gis = [gi0, gi1]
    gos = [go0, go1]

    wid = lax.axis_index("s") * NC + lax.axis_index("c")
    base_b = wid * BTILE

    def in_slab(u):
        off = pl.multiple_of((u * BATCH + base_b) // 2, 64)
        return rows_hbm.at[pl.ds(off, BTILE // 2), :]

    def out_slab(u):
        return out_hbm.at[u, :, pl.ds(pl.multiple_of(base_b, 128), BTILE)]

    def start_in(u, buf, sem):
        pltpu.async_copy(in_slab(u), buf, sem)

    def wait_in(u, buf, sem):
        pltpu.make_async_copy(in_slab(u), buf, sem).wait()

    def start_out(u, tout, sem):
        pltpu.async_copy(tout, out_slab(u), sem)

    def wait_out(u, tout, sem):
        pltpu.make_async_copy(tout, out_slab(u), sem).wait()

    def transpose_blk(buf, tout):
        i16 = lax.iota(jnp.int32, 16)

        @pl.loop(0, BTILE // 16)
        def _(t0_):
            tv = i16 + t0_ * 16
            qv = lax.shift_right_logical(tv, 1)
            hv = lax.shift_left(lax.bitwise_and(tv, 1), 6)
            for k in range(16):
                rot = lax.bitwise_and(i16 + k, 15)
                hrot = hv + rot
                vals = [
                    plsc.load_gather(buf, [qv, hrot + d0 * 16])
                    for d0 in range(DIM // 16)
                ]
                for d0 in range(DIM // 16):
                    plsc.store_scatter(tout, [rot + d0 * 16, tv], vals[d0])

    for j in range(2):
        start_in(j, ains[j], gis[j])

    @pl.loop(0, SEQ, step=2)
    def _ring(s):
        for j in range(2):
            u = s + j
            wait_in(u, ains[j], gis[j])

            @pl.when(u >= 2)
            def _():
                wait_out(u - 2, touts[j], gos[j])

            transpose_blk(ains[j], touts[j])
            start_out(u, touts[j], gos[j])

            @pl.when(u + 2 < SEQ)
            def _():
                start_in(u + 2, ains[j], gis[j])

    for u in (SEQ - 2, SEQ - 1):
        wait_out(u, touts[u % 2], gos[u % 2])


def kernel(idx, table):
    idxT = jnp.transpose(idx)
    rows = _gather(idxT, table)
    kout = _transpose(jnp.reshape(rows, (NTOK // 2, 2 * DIM)))
    return jnp.transpose(kout, (2, 0, 1))


# speedup vs baseline: 1.5619x; 1.3533x over previous
import functools

import jax
import jax.numpy as jnp
from jax import lax
from jax.experimental import pallas as pl
from jax.experimental.pallas import tpu as pltpu
from jax.experimental.pallas import tpu_sc as plsc

DIM = 64
BATCH = 4096
SEQ = 200
VOCAB = 1000000

NC = 2
NS = 16
NW = NC * NS
BTILE = BATCH // NW
NBUF = 4
NTOK = BATCH * SEQ

_mesh = plsc.VectorSubcoreMesh(core_axis_name="c", subcore_axis_name="s")


NBLK = VOCAB // 128
BLK_PER_W = (NBLK + NW - 1) // NW


@functools.partial(
    pl.kernel,
    mesh=_mesh,
    out_type=jax.ShapeDtypeStruct((VOCAB // 2, 2 * DIM), jnp.float32),
    scratch_types=[
        *[pltpu.VMEM((DIM, 128), jnp.float32) for _ in range(2)],
        *[pltpu.VMEM((DIM, 128), jnp.float32) for _ in range(2)],
        pltpu.VMEM((32, 128), jnp.float32),
        *[pltpu.SemaphoreType.DMA for _ in range(2)],
        *[pltpu.SemaphoreType.DMA for _ in range(2)],
    ],
    compiler_params=pltpu.CompilerParams(
        use_tc_tiling_on_sc=True, needs_layout_passes=False),
)
def _relayout(tT_hbm, tail_hbm, tpair_hbm, a0, a1, b0, b1, tailbuf,
              gi0, gi1, go0, go1):
    ains = [a0, a1]
    bouts = [b0, b1]
    gis = [gi0, gi1]
    gos = [go0, go1]

    wid = lax.axis_index("s") * NC + lax.axis_index("c")

    @pl.when(wid == 0)
    def _():
        pltpu.sync_copy(tail_hbm, tailbuf)
        pltpu.sync_copy(tailbuf, tpair_hbm.at[pl.ds(NBLK * 64, 32), :])

    def in_slice(c):
        return tT_hbm.at[:, pl.ds(c * 128, 128)]

    def out_slice(c):
        return tpair_hbm.at[pl.ds(c * 64, 64), :]

    def start_in(c, buf, sem):
        pltpu.async_copy(in_slice(c), buf, sem)

    def wait_in(c, buf, sem):
        pltpu.make_async_copy(in_slice(c), buf, sem).wait()

    def start_out(c, bufp, sem):
        pltpu.async_copy(bufp, out_slice(c), sem)

    def wait_out(c, bufp, sem):
        pltpu.make_async_copy(bufp, out_slice(c), sem).wait()

    def transpose_blk(buf, bufp):
        i16 = lax.iota(jnp.int32, 16)

        @pl.loop(0, 8)
        def _(j0):
            jv = i16 + j0 * 16
            qv = lax.shift_right_logical(jv, 1)
            hv = lax.shift_left(lax.bitwise_and(jv, 1), 6)
            for k in range(16):
                rot = lax.bitwise_and(i16 + k, 15)
                vals = [
                    plsc.load_gather(buf, [rot + d0 * 16, jv])
                    for d0 in range(DIM // 16)
                ]
                for d0 in range(DIM // 16):
                    plsc.store_scatter(bufp, [qv, hv + rot + d0 * 16],
                                       vals[d0])

    def blk(i):
        return wid + NW * i

    for j in range(2):
        @pl.when(blk(j) < NBLK)
        def _():
            start_in(blk(j), ains[j], gis[j])

    @pl.loop(0, BLK_PER_W)
    def _loop(i):
        par_dyn = lax.bitwise_and(i, 1)
        for par in range(2):
            @pl.when(par_dyn == par)
            def _():
                c = blk(i)

                @pl.when(c < NBLK)
                def _():
                    wait_in(c, ains[par], gis[par])

                    @pl.when(i >= 2)
                    def _():
                        wait_out(blk(i - 2), bouts[par], gos[par])

                    transpose_blk(ains[par], bouts[par])
                    start_out(c, bouts[par], gos[par])

                    @pl.when(blk(i + 2) < NBLK)
                    def _():
                        start_in(blk(i + 2), ains[par], gis[par])

    for i_last in (BLK_PER_W - 2, BLK_PER_W - 1):
        par = i_last % 2

        @pl.when(blk(i_last) < NBLK)
        def _():
            wait_out(blk(i_last), bouts[par], gos[par])


@functools.partial(
    pl.kernel,
    mesh=_mesh,
    out_type=jax.ShapeDtypeStruct((NTOK, DIM), jnp.float32),
    scratch_types=[
        pltpu.VMEM((SEQ, BTILE), jnp.int32),
        *[pltpu.VMEM((BTILE, DIM), jnp.float32) for _ in range(NBUF)],
        *[pltpu.VMEM((BTILE,), jnp.int32) for _ in range(NBUF)],
        *[pltpu.SemaphoreType.DMA for _ in range(NBUF)],
        *[pltpu.SemaphoreType.DMA for _ in range(NBUF)],
    ],
    compiler_params=pltpu.CompilerParams(
        use_tc_tiling_on_sc=False, needs_layout_passes=False),
)
def _gather(idxT_hbm, table_hbm, rows_hbm, idxblk,
            b0, b1, b2, b3, p0, p1, p2, p3,
            g0, g1, g2, g3, o0, o1, o2, o3):
    bufs = [b0, b1, b2, b3]
    pids = [p0, p1, p2, p3]
    gsems = [g0, g1, g2, g3]
    osems = [o0, o1, o2, o3]

    wid = lax.axis_index("s") * NC + lax.axis_index("c")
    base_b = wid * BTILE

    pltpu.sync_copy(idxT_hbm.at[:, pl.ds(base_b, BTILE)], idxblk)

    def prep(u, pid):
        @pl.loop(0, BTILE // 16)
        def _(g):
            pid[pl.ds(g * 16, 16)] = idxblk[u, pl.ds(g * 16, 16)]

    def start_gather(buf, pid, sem):
        pltpu.async_copy(table_hbm.at[pid], buf, sem)

    def wait_gather(buf, pid, sem):
        pltpu.make_async_copy(table_hbm.at[pid], buf, sem).wait()

    def out_slab(u):
        return rows_hbm.at[pl.ds(u * BATCH + base_b, BTILE), :]

    def start_out(u, buf, sem):
        pltpu.async_copy(buf, out_slab(u), sem)

    def wait_out(u, buf, sem):
        pltpu.make_async_copy(buf, out_slab(u), sem).wait()

    for j in range(NBUF):
        prep(j, pids[j])
        start_gather(bufs[j], pids[j], gsems[j])

    @pl.loop(0, SEQ, step=NBUF)
    def _ring(s):
        for j in range(NBUF):
            u = s + j
            wait_gather(bufs[j], pids[j], gsems[j])
            start_out(u, bufs[j], osems[j])

            @pl.when(u + NBUF < SEQ)
            def _():
                prep(u + NBUF, pids[j])
                wait_out(u, bufs[j], osems[j])
                start_gather(bufs[j], pids[j], gsems[j])

    for j in range(NBUF):
        wait_out(SEQ - NBUF + j, bufs[j], osems[j])


@functools.partial(
    pl.kernel,
    mesh=_mesh,
    out_type=jax.ShapeDtypeStruct((SEQ, DIM, BATCH), jnp.float32),
    scratch_types=[
        *[pltpu.VMEM((BTILE // 2, 2 * DIM), jnp.float32) for _ in range(2)],
        *[pltpu.VMEM((DIM, BTILE), jnp.float32) for _ in range(2)],
        *[pltpu.SemaphoreType.DMA for _ in range(2)],
        *[pltpu.SemaphoreType.DMA for _ in range(2)],
    ],
    compiler_params=pltpu.CompilerParams(
        use_tc_tiling_on_sc=True, needs_layout_passes=False),
)
def _transpose(rows_hbm, out_hbm, a0, a1, t0, t1, gi0, gi1, go0, go1):
    ains = [a0, a1]
    touts = [t0, t1]
    gis = [gi0, gi1]
    gos = [go0, go1]

    wid = lax.axis_index("s") * NC + lax.axis_index("c")
    base_b = wid * BTILE

    def in_slab(u):
        off = pl.multiple_of((u * BATCH + base_b) // 2, 64)
        return rows_hbm.at[pl.ds(off, BTILE // 2), :]

    def out_slab(u):
        return out_hbm.at[u, :, pl.ds(pl.multiple_of(base_b, 128), BTILE)]

    def start_in(u, buf, sem):
        pltpu.async_copy(in_slab(u), buf, sem)

    def wait_in(u, buf, sem):
        pltpu.make_async_copy(in_slab(u), buf, sem).wait()

    def start_out(u, tout, sem):
        pltpu.async_copy(tout, out_slab(u), sem)

    def wait_out(u, tout, sem):
        pltpu.make_async_copy(tout, out_slab(u), sem).wait()

    def transpose_blk(buf, tout):
        i16 = lax.iota(jnp.int32, 16)

        @pl.loop(0, BTILE // 16)
        def _(t0_):
            tv = i16 + t0_ * 16
            qv = lax.shift_right_logical(tv, 1)
            hv = lax.shift_left(lax.bitwise_and(tv, 1), 6)
            for k in range(16):
                rot = lax.bitwise_and(i16 + k, 15)
                hrot = hv + rot
                vals = [
                    plsc.load_gather(buf, [qv, hrot + d0 * 16])
                    for d0 in range(DIM // 16)
                ]
                for d0 in range(DIM // 16):
                    plsc.store_scatter(tout, [rot + d0 * 16, tv], vals[d0])

    for j in range(2):
        start_in(j, ains[j], gis[j])

    @pl.loop(0, SEQ, step=2)
    def _ring(s):
        for j in range(2):
            u = s + j
            wait_in(u, ains[j], gis[j])

            @pl.when(u >= 2)
            def _():
                wait_out(u - 2, touts[j], gos[j])

            transpose_blk(ains[j], touts[j])
            start_out(u, touts[j], gos[j])

            @pl.when(u + 2 < SEQ)
            def _():
                start_in(u + 2, ains[j], gis[j])

    for u in (SEQ - 2, SEQ - 1):
        wait_out(u, touts[u % 2], gos[u % 2])


def kernel(idx, table):
    idxT = jnp.transpose(idx)
    tail = jax.lax.slice(table, (VOCAB - 64, 0), (VOCAB, DIM))
    tpair = _relayout(jnp.transpose(table),
                      jnp.reshape(tail, (32, 2 * DIM)))
    tlin = jnp.reshape(tpair, (VOCAB, DIM))
    rows = _gather(idxT, tlin)
    kout = _transpose(jnp.reshape(rows, (NTOK // 2, 2 * DIM)))
    return jnp.transpose(kout, (2, 0, 1))
